# baseline (speedup 1.0000x reference)
import functools

import jax
import jax.numpy as jnp
from jax import lax
from jax.experimental import pallas as pl
from jax.experimental.pallas import tpu as pltpu
from jax.experimental.pallas import tpu_sc as plsc

B = 16384
K = 20
D = 32
NW = 32
ROWS_W = B // NW
C = 64
NCHUNK = ROWS_W // C
NIDX_ROWS = C * K // 128
G = C // 16


def _body(uidx_hbm, pidx_hbm, nidx_hbm, utab, itab, pos_out, neg_out,
          uidx_v, pidx_v, nidx_v, urows_v, prows_v, nrows_v, pout_v, nout_v,
          sem_a, sem_b):
    cid = lax.axis_index("c")
    sid = lax.axis_index("s")
    wid = sid * 2 + cid
    l16 = lax.iota(jnp.int32, 16)
    cds = [jnp.full((16,), d, jnp.int32) for d in range(D)]
    sems = [sem_a, sem_b]

    def stage_and_fire(c):
        p = c % 2
        base = wid * ROWS_W + c * C
        pltpu.sync_copy(uidx_hbm.at[pl.ds(base, C)], uidx_v.at[p])
        pltpu.sync_copy(pidx_hbm.at[pl.ds(base, C)], pidx_v.at[p])
        for j in range(NIDX_ROWS):
            pltpu.sync_copy(nidx_hbm.at[pl.ds(base * K + j * 128, 128)],
                            nidx_v.at[p, j])
        hs = [pltpu.async_copy(utab.at[uidx_v.at[p]], urows_v.at[p], sems[p]),
              pltpu.async_copy(itab.at[pidx_v.at[p]], prows_v.at[p], sems[p])]
        for j in range(NIDX_ROWS):
            hs.append(pltpu.async_copy(
                itab.at[nidx_v.at[p, j]],
                nrows_v.at[p, pl.ds(j * 128, 128)], sems[p]))
        return hs

    def compute(c):
        p = c % 2
        base = wid * ROWS_W + c * C
        urows = urows_v.at[p]
        prows = prows_v.at[p]
        nrows = nrows_v.at[p]

        def group_body(g, gcarry):
            rowb = g * 16 + l16
            ucols = [plsc.load_gather(urows, [rowb, cds[d]])
                     for d in range(D)]
            accp0 = ucols[0] * plsc.load_gather(prows, [rowb, cds[0]])
            accp1 = ucols[1] * plsc.load_gather(prows, [rowb, cds[1]])
            for d in range(2, D, 2):
                accp0 = accp0 + ucols[d] * plsc.load_gather(
                    prows, [rowb, cds[d]])
                accp1 = accp1 + ucols[d + 1] * plsc.load_gather(
                    prows, [rowb, cds[d + 1]])
            pout_v[pl.ds(g * 16, 16)] = accp0 + accp1

            def k_body(k4, kcarry):
                for kk in range(4):
                    k = k4 * 4 + kk
                    rowbk = rowb * K + k
                    accn0 = ucols[0] * plsc.load_gather(nrows,
                                                        [rowbk, cds[0]])
                    accn1 = ucols[1] * plsc.load_gather(nrows,
                                                        [rowbk, cds[1]])
                    for d in range(2, D, 2):
                        accn0 = accn0 + ucols[d] * plsc.load_gather(
                            nrows, [rowbk, cds[d]])
                        accn1 = accn1 + ucols[d + 1] * plsc.load_gather(
                            nrows, [rowbk, cds[d + 1]])
                    plsc.store_scatter(
                        nout_v, [rowb, jnp.full((16,), 0, jnp.int32) + k],
                        accn0 + accn1)
                return kcarry

            lax.fori_loop(0, K // 4, k_body, 0)
            return gcarry

        lax.fori_loop(0, G, group_body, 0)
        pltpu.sync_copy(pout_v, pos_out.at[pl.ds(base, C)])
        pltpu.sync_copy(nout_v, neg_out.at[pl.ds(base, C)])

    hs = stage_and_fire(0)
    for c in range(NCHUNK):
        nxt = stage_and_fire(c + 1) if c + 1 < NCHUNK else []
        for h in hs:
            h.wait()
        compute(c)
        hs = nxt


@jax.jit
def _sc_call(user, pos_item, neg_flat, utab, itab):
    mesh = plsc.VectorSubcoreMesh(core_axis_name="c", subcore_axis_name="s")
    kfn = functools.partial(
        pl.kernel,
        out_type=[jax.ShapeDtypeStruct((B,), jnp.float32),
                  jax.ShapeDtypeStruct((B, K), jnp.float32)],
        mesh=mesh,
        scratch_types=[
            pltpu.VMEM((2, C), jnp.int32),
            pltpu.VMEM((2, C), jnp.int32),
            pltpu.VMEM((2, NIDX_ROWS, 128), jnp.int32),
            pltpu.VMEM((2, C, D), jnp.float32),
            pltpu.VMEM((2, C, D), jnp.float32),
            pltpu.VMEM((2, C * K, D), jnp.float32),
            pltpu.VMEM((C,), jnp.float32),
            pltpu.VMEM((C, K), jnp.float32),
            pltpu.SemaphoreType.DMA,
            pltpu.SemaphoreType.DMA,
        ],
        compiler_params=pltpu.CompilerParams(needs_layout_passes=False,
                                             use_tc_tiling_on_sc=False),
    )(_body)
    return kfn(user, pos_item, neg_flat, utab, itab)


def kernel(user, pos_item, neg_items, user_table, item_table):
    user = user.astype(jnp.int32)
    pos_item = pos_item.astype(jnp.int32)
    neg_flat = neg_items.astype(jnp.int32).reshape(B * K)
    pos_s, neg_s = _sc_call(user, pos_item, neg_flat, user_table, item_table)
    return (pos_s, neg_s)


# speedup vs baseline: 1.0141x; 1.0141x over previous
import functools

import jax
import jax.numpy as jnp
from jax import lax
from jax.experimental import pallas as pl
from jax.experimental.pallas import tpu as pltpu
from jax.experimental.pallas import tpu_sc as plsc

B = 16384
K = 20
D = 32
NW = 32
ROWS_W = B // NW
C = 64
NCHUNK = ROWS_W // C
NIDX_ROWS = C * K // 128
G = C // 16


def _body(uidx_hbm, pidx_hbm, nidx_hbm, utab, itab, pos_out, neg_out,
          uidx_v, pidx_v, nidx_v, urows_v, prows_v, nrows_v, pout_v, nout_v,
          sem_a, sem_b):
    cid = lax.axis_index("c")
    sid = lax.axis_index("s")
    wid = sid * 2 + cid
    l16 = lax.iota(jnp.int32, 16)
    cds = [jnp.full((16,), d, jnp.int32) for d in range(D)]
    sems = [sem_a, sem_b]

    def stage_and_fire(c):
        p = c % 2
        base = wid * ROWS_W + c * C
        pltpu.sync_copy(uidx_hbm.at[pl.ds(base, C)], uidx_v.at[p])
        pltpu.sync_copy(pidx_hbm.at[pl.ds(base, C)], pidx_v.at[p])
        for j in range(NIDX_ROWS):
            pltpu.sync_copy(nidx_hbm.at[pl.ds(base * K + j * 128, 128)],
                            nidx_v.at[p, j])
        hs = [pltpu.async_copy(utab.at[uidx_v.at[p]], urows_v.at[p], sems[p]),
              pltpu.async_copy(itab.at[pidx_v.at[p]], prows_v.at[p], sems[p])]
        for j in range(NIDX_ROWS):
            hs.append(pltpu.async_copy(
                itab.at[nidx_v.at[p, j]],
                nrows_v.at[p, pl.ds(j * 128, 128)], sems[p]))
        return hs

    lane15 = l16 == 15
    zero16 = jnp.full((16,), 0, jnp.int32)

    def compute(c):
        p = c % 2
        base = wid * ROWS_W + c * C
        urows = urows_v.at[p]
        prows = prows_v.at[p]
        nrows = nrows_v.at[p]

        def row_body(r, rcarry):
            u0 = urows[r, pl.ds(0, 16)]
            u1 = urows[r, pl.ds(16, 16)]
            p0 = prows[r, pl.ds(0, 16)]
            p1 = prows[r, pl.ds(16, 16)]
            ridx = zero16 + r
            plsc.store_scatter(pout_v, [ridx],
                               plsc.cumsum(u0 * p0 + u1 * p1), mask=lane15)
            nr = r * K
            for k in range(K):
                n0 = nrows[nr + k, pl.ds(0, 16)]
                n1 = nrows[nr + k, pl.ds(16, 16)]
                plsc.store_scatter(
                    nout_v, [ridx, zero16 + k],
                    plsc.cumsum(u0 * n0 + u1 * n1), mask=lane15)
            return rcarry

        lax.fori_loop(0, C, row_body, 0)
        pltpu.sync_copy(pout_v, pos_out.at[pl.ds(base, C)])
        pltpu.sync_copy(nout_v, neg_out.at[pl.ds(base, C)])

    hs = stage_and_fire(0)
    for c in range(NCHUNK):
        nxt = stage_and_fire(c + 1) if c + 1 < NCHUNK else []
        for h in hs:
            h.wait()
        compute(c)
        hs = nxt


@jax.jit
def _sc_call(user, pos_item, neg_flat, utab, itab):
    mesh = plsc.VectorSubcoreMesh(core_axis_name="c", subcore_axis_name="s")
    kfn = functools.partial(
        pl.kernel,
        out_type=[jax.ShapeDtypeStruct((B,), jnp.float32),
                  jax.ShapeDtypeStruct((B, K), jnp.float32)],
        mesh=mesh,
        scratch_types=[
            pltpu.VMEM((2, C), jnp.int32),
            pltpu.VMEM((2, C), jnp.int32),
            pltpu.VMEM((2, NIDX_ROWS, 128), jnp.int32),
            pltpu.VMEM((2, C, D), jnp.float32),
            pltpu.VMEM((2, C, D), jnp.float32),
            pltpu.VMEM((2, C * K, D), jnp.float32),
            pltpu.VMEM((C,), jnp.float32),
            pltpu.VMEM((C, K), jnp.float32),
            pltpu.SemaphoreType.DMA,
            pltpu.SemaphoreType.DMA,
        ],
        compiler_params=pltpu.CompilerParams(needs_layout_passes=False,
                                             use_tc_tiling_on_sc=False),
    )(_body)
    return kfn(user, pos_item, neg_flat, utab, itab)


def kernel(user, pos_item, neg_items, user_table, item_table):
    user = user.astype(jnp.int32)
    pos_item = pos_item.astype(jnp.int32)
    neg_flat = neg_items.astype(jnp.int32).reshape(B * K)
    pos_s, neg_s = _sc_call(user, pos_item, neg_flat, user_table, item_table)
    return (pos_s, neg_s)


# speedup vs baseline: 1.0504x; 1.0358x over previous
import functools

import jax
import jax.numpy as jnp
from jax import lax
from jax.experimental import pallas as pl
from jax.experimental.pallas import tpu as pltpu
from jax.experimental.pallas import tpu_sc as plsc

B = 16384
K = 20
D = 32
NW = 32
ROWS_W = B // NW
C = 64
NCHUNK = ROWS_W // C
NIDX_ROWS = C * K // 128
G = C // 16


def _body(uidx_hbm, pidx_hbm, nidx_hbm, utab, itab, pos_out, neg_out,
          uidx_v, pidx_v, nidx_v, urows_v, prows_v, nrows_v, pout_v, nout_v,
          sem_a, sem_b, sem_i):
    cid = lax.axis_index("c")
    sid = lax.axis_index("s")
    wid = sid * 2 + cid
    l16 = lax.iota(jnp.int32, 16)
    cds = [jnp.full((16,), d, jnp.int32) for d in range(D)]
    sems = [sem_a, sem_b]

    def stage_and_fire(c):
        p = c % 2
        base = wid * ROWS_W + c * C
        ih = [pltpu.async_copy(uidx_hbm.at[pl.ds(base, C)], uidx_v.at[p],
                               sem_i),
              pltpu.async_copy(pidx_hbm.at[pl.ds(base, C)], pidx_v.at[p],
                               sem_i),
              pltpu.async_copy(nidx_hbm.at[pl.ds(base * K, C * K)],
                               nidx_v.at[p], sem_i)]
        for h in ih:
            h.wait()
        hs = [pltpu.async_copy(utab.at[uidx_v.at[p]], urows_v.at[p], sems[p]),
              pltpu.async_copy(itab.at[pidx_v.at[p]], prows_v.at[p], sems[p])]
        for j in range(NIDX_ROWS):
            hs.append(pltpu.async_copy(
                itab.at[nidx_v.at[p, pl.ds(j * 128, 128)]],
                nrows_v.at[p, pl.ds(j * 128, 128)], sems[p]))
        return hs

    lane15 = l16 == 15
    zero16 = jnp.full((16,), 0, jnp.int32)

    def compute(c):
        p = c % 2
        base = wid * ROWS_W + c * C
        urows = urows_v.at[p]
        prows = prows_v.at[p]
        nrows = nrows_v.at[p]

        def row_body(r, rcarry):
            u0 = urows[r, pl.ds(0, 16)]
            u1 = urows[r, pl.ds(16, 16)]
            p0 = prows[r, pl.ds(0, 16)]
            p1 = prows[r, pl.ds(16, 16)]
            ridx = zero16 + r
            plsc.store_scatter(pout_v, [ridx],
                               plsc.cumsum(u0 * p0 + u1 * p1), mask=lane15)
            nr = r * K
            for k in range(K):
                n0 = nrows[nr + k, pl.ds(0, 16)]
                n1 = nrows[nr + k, pl.ds(16, 16)]
                plsc.store_scatter(
                    nout_v, [ridx, zero16 + k],
                    plsc.cumsum(u0 * n0 + u1 * n1), mask=lane15)
            return rcarry

        lax.fori_loop(0, C, row_body, 0)
        pltpu.sync_copy(pout_v, pos_out.at[pl.ds(base, C)])
        pltpu.sync_copy(nout_v, neg_out.at[pl.ds(base, C)])

    hs = stage_and_fire(0)
    for c in range(NCHUNK):
        nxt = stage_and_fire(c + 1) if c + 1 < NCHUNK else []
        for h in hs:
            h.wait()
        compute(c)
        hs = nxt


@jax.jit
def _sc_call(user, pos_item, neg_flat, utab, itab):
    mesh = plsc.VectorSubcoreMesh(core_axis_name="c", subcore_axis_name="s")
    kfn = functools.partial(
        pl.kernel,
        out_type=[jax.ShapeDtypeStruct((B,), jnp.float32),
                  jax.ShapeDtypeStruct((B, K), jnp.float32)],
        mesh=mesh,
        scratch_types=[
            pltpu.VMEM((2, C), jnp.int32),
            pltpu.VMEM((2, C), jnp.int32),
            pltpu.VMEM((2, C * K), jnp.int32),
            pltpu.VMEM((2, C, D), jnp.float32),
            pltpu.VMEM((2, C, D), jnp.float32),
            pltpu.VMEM((2, C * K, D), jnp.float32),
            pltpu.VMEM((C,), jnp.float32),
            pltpu.VMEM((C, K), jnp.float32),
            pltpu.SemaphoreType.DMA,
            pltpu.SemaphoreType.DMA,
            pltpu.SemaphoreType.DMA,
        ],
        compiler_params=pltpu.CompilerParams(needs_layout_passes=False,
                                             use_tc_tiling_on_sc=False),
    )(_body)
    return kfn(user, pos_item, neg_flat, utab, itab)


def kernel(user, pos_item, neg_items, user_table, item_table):
    user = user.astype(jnp.int32)
    pos_item = pos_item.astype(jnp.int32)
    neg_flat = neg_items.astype(jnp.int32).reshape(B * K)
    pos_s, neg_s = _sc_call(user, pos_item, neg_flat, user_table, item_table)
    return (pos_s, neg_s)


# speedup vs baseline: 1.1176x; 1.0640x over previous
import functools

import jax
import jax.numpy as jnp
from jax import lax
from jax.experimental import pallas as pl
from jax.experimental.pallas import tpu as pltpu
from jax.experimental.pallas import tpu_sc as plsc

B = 16384
K = 20
D = 32
NW = 32
ROWS_W = B // NW
C = 64
NCHUNK = ROWS_W // C
NIDX_ROWS = C * K // 128
G = C // 16


def _body(uidx_hbm, pidx_hbm, nidx_hbm, utab, itab, pos_out, neg_out,
          uidx_v, pidx_v, nidx_v, urows_v, prows_v, nrows_v, pout_v, nout_v,
          sem_a, sem_b, sem_i):
    cid = lax.axis_index("c")
    sid = lax.axis_index("s")
    wid = sid * 2 + cid
    l16 = lax.iota(jnp.int32, 16)
    cds = [jnp.full((16,), d, jnp.int32) for d in range(D)]
    sems = [sem_a, sem_b]

    wbase = wid * ROWS_W
    ih = [pltpu.async_copy(uidx_hbm.at[pl.ds(wbase, ROWS_W)], uidx_v, sem_i),
          pltpu.async_copy(pidx_hbm.at[pl.ds(wbase, ROWS_W)], pidx_v, sem_i),
          pltpu.async_copy(nidx_hbm.at[pl.ds(wbase * K, ROWS_W * K)],
                           nidx_v, sem_i)]
    for h in ih:
        h.wait()

    def stage_and_fire(c):
        p = c % 2
        hs = [pltpu.async_copy(utab.at[uidx_v.at[pl.ds(c * C, C)]],
                               urows_v.at[p], sems[p]),
              pltpu.async_copy(itab.at[pidx_v.at[pl.ds(c * C, C)]],
                               prows_v.at[p], sems[p])]
        for j in range(NIDX_ROWS):
            hs.append(pltpu.async_copy(
                itab.at[nidx_v.at[pl.ds(c * C * K + j * 128, 128)]],
                nrows_v.at[p, pl.ds(j * 128, 128)], sems[p]))
        return hs

    lane15 = l16 == 15
    zero16 = jnp.full((16,), 0, jnp.int32)

    def compute(c):
        p = c % 2
        base = wid * ROWS_W + c * C
        urows = urows_v.at[p]
        prows = prows_v.at[p]
        nrows = nrows_v.at[p]

        def row_body(r2, rcarry):
            ra = r2 * 2
            rb = ra + 1
            ua0 = urows[ra, pl.ds(0, 16)]
            ua1 = urows[ra, pl.ds(16, 16)]
            ub0 = urows[rb, pl.ds(0, 16)]
            ub1 = urows[rb, pl.ds(16, 16)]
            pa0 = prows[ra, pl.ds(0, 16)]
            pa1 = prows[ra, pl.ds(16, 16)]
            pb0 = prows[rb, pl.ds(0, 16)]
            pb1 = prows[rb, pl.ds(16, 16)]
            ridxa = zero16 + ra
            ridxb = zero16 + rb
            plsc.store_scatter(pout_v, [ridxa],
                               plsc.cumsum(ua0 * pa0 + ua1 * pa1),
                               mask=lane15)
            plsc.store_scatter(pout_v, [ridxb],
                               plsc.cumsum(ub0 * pb0 + ub1 * pb1),
                               mask=lane15)
            nra = ra * K
            nrb = rb * K
            for k in range(K):
                na0 = nrows[nra + k, pl.ds(0, 16)]
                na1 = nrows[nra + k, pl.ds(16, 16)]
                nb0 = nrows[nrb + k, pl.ds(0, 16)]
                nb1 = nrows[nrb + k, pl.ds(16, 16)]
                ca = plsc.cumsum(ua0 * na0 + ua1 * na1)
                cb = plsc.cumsum(ub0 * nb0 + ub1 * nb1)
                plsc.store_scatter(nout_v, [ridxa, zero16 + k], ca,
                                   mask=lane15)
                plsc.store_scatter(nout_v, [ridxb, zero16 + k], cb,
                                   mask=lane15)
            return rcarry

        lax.fori_loop(0, C // 2, row_body, 0)
        pltpu.sync_copy(pout_v, pos_out.at[pl.ds(base, C)])
        pltpu.sync_copy(nout_v, neg_out.at[pl.ds(base, C)])

    hs = stage_and_fire(0)
    for c in range(NCHUNK):
        nxt = stage_and_fire(c + 1) if c + 1 < NCHUNK else []
        for h in hs:
            h.wait()
        compute(c)
        hs = nxt


@jax.jit
def _sc_call(user, pos_item, neg_flat, utab, itab):
    mesh = plsc.VectorSubcoreMesh(core_axis_name="c", subcore_axis_name="s")
    kfn = functools.partial(
        pl.kernel,
        out_type=[jax.ShapeDtypeStruct((B,), jnp.float32),
                  jax.ShapeDtypeStruct((B, K), jnp.float32)],
        mesh=mesh,
        scratch_types=[
            pltpu.VMEM((ROWS_W,), jnp.int32),
            pltpu.VMEM((ROWS_W,), jnp.int32),
            pltpu.VMEM((ROWS_W * K,), jnp.int32),
            pltpu.VMEM((2, C, D), jnp.float32),
            pltpu.VMEM((2, C, D), jnp.float32),
            pltpu.VMEM((2, C * K, D), jnp.float32),
            pltpu.VMEM((C,), jnp.float32),
            pltpu.VMEM((C, K), jnp.float32),
            pltpu.SemaphoreType.DMA,
            pltpu.SemaphoreType.DMA,
            pltpu.SemaphoreType.DMA,
        ],
        compiler_params=pltpu.CompilerParams(needs_layout_passes=False,
                                             use_tc_tiling_on_sc=False),
    )(_body)
    return kfn(user, pos_item, neg_flat, utab, itab)


def kernel(user, pos_item, neg_items, user_table, item_table):
    user = user.astype(jnp.int32)
    pos_item = pos_item.astype(jnp.int32)
    neg_flat = neg_items.astype(jnp.int32).reshape(B * K)
    pos_s, neg_s = _sc_call(user, pos_item, neg_flat, user_table, item_table)
    return (pos_s, neg_s)


# speedup vs baseline: 1.1494x; 1.0284x over previous
import functools

import jax
import jax.numpy as jnp
from jax import lax
from jax.experimental import pallas as pl
from jax.experimental.pallas import tpu as pltpu
from jax.experimental.pallas import tpu_sc as plsc

B = 16384
K = 20
D = 32
NW = 32
ROWS_W = B // NW
C = 64
NCHUNK = ROWS_W // C
NIDX_ROWS = C * K // 128
G = C // 16


def _body(uidx_hbm, pidx_hbm, nidx_hbm, utab, itab, pos_out, neg_out,
          uidx_v, pidx_v, nidx_v, urows_v, prows_v, nrows_v, pout_v, nout_v,
          sem_a, sem_b, sem_i):
    cid = lax.axis_index("c")
    sid = lax.axis_index("s")
    wid = sid * 2 + cid
    l16 = lax.iota(jnp.int32, 16)
    cds = [jnp.full((16,), d, jnp.int32) for d in range(D)]
    sems = [sem_a, sem_b]

    wbase = wid * ROWS_W
    ih = [pltpu.async_copy(uidx_hbm.at[pl.ds(wbase, ROWS_W)], uidx_v, sem_i),
          pltpu.async_copy(pidx_hbm.at[pl.ds(wbase, ROWS_W)], pidx_v, sem_i),
          pltpu.async_copy(nidx_hbm.at[pl.ds(wbase * K, ROWS_W * K)],
                           nidx_v, sem_i)]
    for h in ih:
        h.wait()

    def stage_and_fire(c):
        p = c % 2
        hs = [pltpu.async_copy(utab.at[uidx_v.at[pl.ds(c * C, C)]],
                               urows_v.at[p], sems[p]),
              pltpu.async_copy(itab.at[pidx_v.at[pl.ds(c * C, C)]],
                               prows_v.at[p], sems[p])]
        for j in range(NIDX_ROWS):
            hs.append(pltpu.async_copy(
                itab.at[nidx_v.at[pl.ds(c * C * K + j * 128, 128)]],
                nrows_v.at[p, pl.ds(j * 128, 128)], sems[p]))
        return hs

    lane15 = l16 == 15
    zero16 = jnp.full((16,), 0, jnp.int32)

    def compute(c):
        p = c % 2
        base = wid * ROWS_W + c * C
        urows = urows_v.at[p]
        prows = prows_v.at[p]
        nrows = nrows_v.at[p]

        NI = 4

        def row_body(rq, rcarry):
            rs = [rq * NI + i for i in range(NI)]
            us = [(urows[r, pl.ds(0, 16)], urows[r, pl.ds(16, 16)])
                  for r in rs]
            ps = [(prows[r, pl.ds(0, 16)], prows[r, pl.ds(16, 16)])
                  for r in rs]
            ridx = [zero16 + r for r in rs]
            cps = [plsc.cumsum(us[i][0] * ps[i][0] + us[i][1] * ps[i][1])
                   for i in range(NI)]
            for i in range(NI):
                plsc.store_scatter(pout_v, [ridx[i]], cps[i], mask=lane15)
            for k in range(K):
                ns = [(nrows[rs[i] * K + k, pl.ds(0, 16)],
                       nrows[rs[i] * K + k, pl.ds(16, 16)])
                      for i in range(NI)]
                cs = [plsc.cumsum(us[i][0] * ns[i][0] + us[i][1] * ns[i][1])
                      for i in range(NI)]
                for i in range(NI):
                    plsc.store_scatter(nout_v, [ridx[i], zero16 + k], cs[i],
                                       mask=lane15)
            return rcarry

        lax.fori_loop(0, C // NI, row_body, 0)
        pltpu.sync_copy(pout_v, pos_out.at[pl.ds(base, C)])
        pltpu.sync_copy(nout_v, neg_out.at[pl.ds(base, C)])

    hs = stage_and_fire(0)
    for c in range(NCHUNK):
        nxt = stage_and_fire(c + 1) if c + 1 < NCHUNK else []
        for h in hs:
            h.wait()
        compute(c)
        hs = nxt


@jax.jit
def _sc_call(user, pos_item, neg_flat, utab, itab):
    mesh = plsc.VectorSubcoreMesh(core_axis_name="c", subcore_axis_name="s")
    kfn = functools.partial(
        pl.kernel,
        out_type=[jax.ShapeDtypeStruct((B,), jnp.float32),
                  jax.ShapeDtypeStruct((B, K), jnp.float32)],
        mesh=mesh,
        scratch_types=[
            pltpu.VMEM((ROWS_W,), jnp.int32),
            pltpu.VMEM((ROWS_W,), jnp.int32),
            pltpu.VMEM((ROWS_W * K,), jnp.int32),
            pltpu.VMEM((2, C, D), jnp.float32),
            pltpu.VMEM((2, C, D), jnp.float32),
            pltpu.VMEM((2, C * K, D), jnp.float32),
            pltpu.VMEM((C,), jnp.float32),
            pltpu.VMEM((C, K), jnp.float32),
            pltpu.SemaphoreType.DMA,
            pltpu.SemaphoreType.DMA,
            pltpu.SemaphoreType.DMA,
        ],
        compiler_params=pltpu.CompilerParams(needs_layout_passes=False,
                                             use_tc_tiling_on_sc=False),
    )(_body)
    return kfn(user, pos_item, neg_flat, utab, itab)


def kernel(user, pos_item, neg_items, user_table, item_table):
    user = user.astype(jnp.int32)
    pos_item = pos_item.astype(jnp.int32)
    neg_flat = neg_items.astype(jnp.int32).reshape(B * K)
    pos_s, neg_s = _sc_call(user, pos_item, neg_flat, user_table, item_table)
    return (pos_s, neg_s)


# speedup vs baseline: 1.1661x; 1.0145x over previous
import functools

import jax
import jax.numpy as jnp
from jax import lax
from jax.experimental import pallas as pl
from jax.experimental.pallas import tpu as pltpu
from jax.experimental.pallas import tpu_sc as plsc

B = 16384
K = 20
D = 32
NW = 32
ROWS_W = B // NW
C = 64
NCHUNK = ROWS_W // C
NIDX_ROWS = C * K // 128
G = C // 16


def _body(uidx_hbm, pidx_hbm, nidx_hbm, utab, itab, pos_out, neg_out,
          uidx_v, pidx_v, nidx_v, urows_v, prows_v, nrows_v, pout_v, nout_v,
          sem_a, sem_b, sem_i):
    cid = lax.axis_index("c")
    sid = lax.axis_index("s")
    wid = sid * 2 + cid
    l16 = lax.iota(jnp.int32, 16)
    cds = [jnp.full((16,), d, jnp.int32) for d in range(D)]
    sems = [sem_a, sem_b]

    wbase = wid * ROWS_W
    ih = [pltpu.async_copy(uidx_hbm.at[pl.ds(wbase, ROWS_W)], uidx_v, sem_i),
          pltpu.async_copy(pidx_hbm.at[pl.ds(wbase, ROWS_W)], pidx_v, sem_i),
          pltpu.async_copy(nidx_hbm.at[pl.ds(wbase * K, ROWS_W * K)],
                           nidx_v, sem_i)]
    for h in ih:
        h.wait()

    def stage_and_fire(c):
        p = c % 2
        hs = [pltpu.async_copy(utab.at[uidx_v.at[pl.ds(c * C, C)]],
                               urows_v.at[p], sems[p]),
              pltpu.async_copy(itab.at[pidx_v.at[pl.ds(c * C, C)]],
                               prows_v.at[p], sems[p])]
        for j in range(NIDX_ROWS):
            hs.append(pltpu.async_copy(
                itab.at[nidx_v.at[pl.ds(c * C * K + j * 128, 128)]],
                nrows_v.at[p, pl.ds(j * 128, 128)], sems[p]))
        return hs

    lane15 = l16 == 15
    zero16 = jnp.full((16,), 0, jnp.int32)

    def compute(c):
        p = c % 2
        base = wid * ROWS_W + c * C
        urows = urows_v.at[p]
        prows = prows_v.at[p]
        nrows = nrows_v.at[p]

        NI = 8

        def row_body(rq, rcarry):
            rs = [rq * NI + i for i in range(NI)]
            us = [(urows[r, pl.ds(0, 16)], urows[r, pl.ds(16, 16)])
                  for r in rs]
            ps = [(prows[r, pl.ds(0, 16)], prows[r, pl.ds(16, 16)])
                  for r in rs]
            ridx = [zero16 + r for r in rs]
            cps = [plsc.cumsum(us[i][0] * ps[i][0] + us[i][1] * ps[i][1])
                   for i in range(NI)]
            for i in range(NI):
                plsc.store_scatter(pout_v, [ridx[i]], cps[i], mask=lane15)
            for k in range(K):
                ns = [(nrows[rs[i] * K + k, pl.ds(0, 16)],
                       nrows[rs[i] * K + k, pl.ds(16, 16)])
                      for i in range(NI)]
                cs = [plsc.cumsum(us[i][0] * ns[i][0] + us[i][1] * ns[i][1])
                      for i in range(NI)]
                for i in range(NI):
                    plsc.store_scatter(nout_v, [ridx[i], zero16 + k], cs[i],
                                       mask=lane15)
            return rcarry

        lax.fori_loop(0, C // NI, row_body, 0)
        pltpu.sync_copy(pout_v, pos_out.at[pl.ds(base, C)])
        pltpu.sync_copy(nout_v, neg_out.at[pl.ds(base, C)])

    hs = stage_and_fire(0)
    for c in range(NCHUNK):
        nxt = stage_and_fire(c + 1) if c + 1 < NCHUNK else []
        for h in hs:
            h.wait()
        compute(c)
        hs = nxt


@jax.jit
def _sc_call(user, pos_item, neg_flat, utab, itab):
    mesh = plsc.VectorSubcoreMesh(core_axis_name="c", subcore_axis_name="s")
    kfn = functools.partial(
        pl.kernel,
        out_type=[jax.ShapeDtypeStruct((B,), jnp.float32),
                  jax.ShapeDtypeStruct((B, K), jnp.float32)],
        mesh=mesh,
        scratch_types=[
            pltpu.VMEM((ROWS_W,), jnp.int32),
            pltpu.VMEM((ROWS_W,), jnp.int32),
            pltpu.VMEM((ROWS_W * K,), jnp.int32),
            pltpu.VMEM((2, C, D), jnp.float32),
            pltpu.VMEM((2, C, D), jnp.float32),
            pltpu.VMEM((2, C * K, D), jnp.float32),
            pltpu.VMEM((C,), jnp.float32),
            pltpu.VMEM((C, K), jnp.float32),
            pltpu.SemaphoreType.DMA,
            pltpu.SemaphoreType.DMA,
            pltpu.SemaphoreType.DMA,
        ],
        compiler_params=pltpu.CompilerParams(needs_layout_passes=False,
                                             use_tc_tiling_on_sc=False),
    )(_body)
    return kfn(user, pos_item, neg_flat, utab, itab)


def kernel(user, pos_item, neg_items, user_table, item_table):
    user = user.astype(jnp.int32)
    pos_item = pos_item.astype(jnp.int32)
    neg_flat = neg_items.astype(jnp.int32).reshape(B * K)
    pos_s, neg_s = _sc_call(user, pos_item, neg_flat, user_table, item_table)
    return (pos_s, neg_s)
